# baseline (speedup 1.0000x reference)
import functools
import math

import jax
import jax.numpy as jnp
from jax import lax
from jax.experimental import pallas as pl
from jax.experimental.pallas import tpu as pltpu
from jax.experimental.pallas import tpu_sc as plsc

_DIM = 32
_EMB_SCALE = math.sqrt(float(_DIM))
_NC, _NS, _L = 2, 16, 16
_NW = _NC * _NS
_CH = 128
_NBUF = 8
_V = 1000000
_NT = _V // 128
_VTAIL = _V - _NT * 128


def _transpose_kernel():
  mesh = plsc.VectorSubcoreMesh(core_axis_name="c", subcore_axis_name="s")

  @functools.partial(
      pl.kernel,
      out_type=jax.ShapeDtypeStruct((_V // 4, 128), jnp.float32),
      mesh=mesh,
      compiler_params=pltpu.CompilerParams(use_tc_tiling_on_sc=True,
                                           needs_layout_passes=False),
      scratch_types=[
          pltpu.VMEM((4, 8, 128), jnp.float32),
          pltpu.VMEM((32, 128), jnp.float32),
      ],
  )
  def body(tt_hbm, out_hbm, in_v, outs_v):
    wid = lax.axis_index("s") * _NC + lax.axis_index("c")
    nk = (_NT - wid + _NW - 1) // _NW

    def _lines_loop(nj):
      @pl.loop(0, nj)
      def _lines(jj):
        for lg in range(8):
          d = lax.iota(jnp.int32, _L) + (16 * (lg & 1))
          dt = lax.shift_right_logical(d, 3)
          dm = lax.bitwise_and(d, 7)
          col = jnp.zeros((_L,), jnp.int32) + (4 * jj + (lg >> 1))
          v = plsc.load_gather(in_v, [dt, dm, col])
          outs_v[jj, pl.ds(lg * 16, _L)] = v * _EMB_SCALE

    @pl.loop(0, nk)
    def _tiles(k):
      t = wid + k * _NW
      for dt in range(4):
        pltpu.sync_copy(
            tt_hbm.at[pl.ds(dt * 8, 8), pl.ds(t * 128, 128)], in_v.at[dt])
      _lines_loop(32)
      pltpu.sync_copy(outs_v, out_hbm.at[pl.ds(t * 32, 32)])

  return body


def _gather_kernel(nch):
  mesh = plsc.VectorSubcoreMesh(core_axis_name="c", subcore_axis_name="s")

  @functools.partial(
      pl.kernel,
      out_type=jax.ShapeDtypeStruct((_NW * nch * _CH, _DIM), jnp.float32),
      mesh=mesh,
      compiler_params=pltpu.CompilerParams(use_tc_tiling_on_sc=False),
      scratch_types=[
          pltpu.VMEM((nch, _CH), jnp.int32),
          pltpu.VMEM((_NBUF, _CH, _DIM), jnp.float32),
          [pltpu.SemaphoreType.DMA] * _NBUF,
          [pltpu.SemaphoreType.DMA] * _NBUF,
      ],
  )
  def body(x_hbm, table_hbm, out_hbm, idx_v, rows_v, gsems, osems):
    wid = lax.axis_index("s") * _NC + lax.axis_index("c")
    och = _CH
    obase = wid * (nch * och)
    pltpu.sync_copy(x_hbm.at[wid], idx_v)

    @pl.loop(0, nch, step=_NBUF)
    def _group(g):
      for b in range(_NBUF):
        @pl.when(g > 0)
        def _drain():
          pltpu.make_async_copy(
              rows_v.at[b],
              out_hbm.at[pl.ds(obase + (g - _NBUF + b) * och, och)],
              osems[b]).wait()
        pltpu.async_copy(
            table_hbm.at[idx_v.at[g + b]], rows_v.at[b], gsems[b])
      for b in range(_NBUF):
        c = g + b
        pltpu.make_async_copy(
            table_hbm.at[idx_v.at[c]], rows_v.at[b], gsems[b]).wait()
        pltpu.async_copy(
            rows_v.at[b], out_hbm.at[pl.ds(obase + c * och, och)], osems[b])

    for b in range(_NBUF):
      pltpu.make_async_copy(
          rows_v.at[b],
          out_hbm.at[pl.ds(obase + (nch - _NBUF + b) * och, och)],
          osems[b]).wait()

  return body


def kernel(x, table):
  bsz, seq = x.shape
  tot = bsz * seq
  nch = tot // (_NW * _CH)
  xr = x.astype(jnp.int32).reshape(_NW, nch, _CH)
  t128 = _transpose_kernel()(table.T)
  tail = (table[_NT * 128:] * _EMB_SCALE).reshape(_VTAIL * _DIM // 128, 128)
  t128 = jax.lax.dynamic_update_slice(t128, tail, (_NT * 32, 0))
  tlin = t128.reshape(_V, _DIM)
  out = _gather_kernel(nch)(xr, tlin)
  return out.reshape(bsz, seq, _DIM)


# speedup vs baseline: 1.0005x; 1.0005x over previous
import functools
import math

import jax
import jax.numpy as jnp
from jax import lax
from jax.experimental import pallas as pl
from jax.experimental.pallas import tpu as pltpu
from jax.experimental.pallas import tpu_sc as plsc

_DIM = 32
_EMB_SCALE = math.sqrt(float(_DIM))
_NC, _NS, _L = 2, 16, 16
_NW = _NC * _NS
_CH = 128
_NBUF = 8
_V = 1000000
_NT = _V // 128
_VTAIL = _V - _NT * 128


def _transpose_kernel():
  mesh = plsc.VectorSubcoreMesh(core_axis_name="c", subcore_axis_name="s")

  @functools.partial(
      pl.kernel,
      out_type=jax.ShapeDtypeStruct((_V // 4, 128), jnp.float32),
      mesh=mesh,
      compiler_params=pltpu.CompilerParams(use_tc_tiling_on_sc=True,
                                           needs_layout_passes=False),
      scratch_types=[
          pltpu.VMEM((4, 8, 129), jnp.float32),
          pltpu.VMEM((32, 128), jnp.float32),
      ],
  )
  def body(tt_hbm, out_hbm, in_v, outs_v):
    wid = lax.axis_index("s") * _NC + lax.axis_index("c")
    nk = (_NT - wid + _NW - 1) // _NW

    def _lines_loop(nj):
      @pl.loop(0, nj)
      def _lines(jj):
        for lg in range(8):
          d = lax.iota(jnp.int32, _L) + (16 * (lg & 1))
          dt = lax.shift_right_logical(d, 3)
          dm = lax.bitwise_and(d, 7)
          col = jnp.zeros((_L,), jnp.int32) + (4 * jj + (lg >> 1))
          v = plsc.load_gather(in_v, [dt, dm, col])
          outs_v[jj, pl.ds(lg * 16, _L)] = v * _EMB_SCALE

    @pl.loop(0, nk)
    def _tiles(k):
      t = wid + k * _NW
      for dt in range(4):
        pltpu.sync_copy(
            tt_hbm.at[pl.ds(dt * 8, 8), pl.ds(t * 128, 128)],
            in_v.at[dt, :, pl.ds(0, 128)])
      _lines_loop(32)
      pltpu.sync_copy(outs_v, out_hbm.at[pl.ds(t * 32, 32)])

  return body


def _gather_kernel(nch):
  mesh = plsc.VectorSubcoreMesh(core_axis_name="c", subcore_axis_name="s")

  @functools.partial(
      pl.kernel,
      out_type=jax.ShapeDtypeStruct((_NW * nch * _CH, _DIM), jnp.float32),
      mesh=mesh,
      compiler_params=pltpu.CompilerParams(use_tc_tiling_on_sc=False),
      scratch_types=[
          pltpu.VMEM((nch, _CH), jnp.int32),
          pltpu.VMEM((_NBUF, _CH, _DIM), jnp.float32),
          [pltpu.SemaphoreType.DMA] * _NBUF,
          [pltpu.SemaphoreType.DMA] * _NBUF,
      ],
  )
  def body(x_hbm, table_hbm, out_hbm, idx_v, rows_v, gsems, osems):
    wid = lax.axis_index("s") * _NC + lax.axis_index("c")
    och = _CH
    obase = wid * (nch * och)
    pltpu.sync_copy(x_hbm.at[wid], idx_v)

    @pl.loop(0, nch, step=_NBUF)
    def _group(g):
      for b in range(_NBUF):
        @pl.when(g > 0)
        def _drain():
          pltpu.make_async_copy(
              rows_v.at[b],
              out_hbm.at[pl.ds(obase + (g - _NBUF + b) * och, och)],
              osems[b]).wait()
        pltpu.async_copy(
            table_hbm.at[idx_v.at[g + b]], rows_v.at[b], gsems[b])
      for b in range(_NBUF):
        c = g + b
        pltpu.make_async_copy(
            table_hbm.at[idx_v.at[c]], rows_v.at[b], gsems[b]).wait()
        pltpu.async_copy(
            rows_v.at[b], out_hbm.at[pl.ds(obase + c * och, och)], osems[b])

    for b in range(_NBUF):
      pltpu.make_async_copy(
          rows_v.at[b],
          out_hbm.at[pl.ds(obase + (nch - _NBUF + b) * och, och)],
          osems[b]).wait()

  return body


def kernel(x, table):
  bsz, seq = x.shape
  tot = bsz * seq
  nch = tot // (_NW * _CH)
  xr = x.astype(jnp.int32).reshape(_NW, nch, _CH)
  t128 = _transpose_kernel()(table.T)
  tail = (table[_NT * 128:] * _EMB_SCALE).reshape(_VTAIL * _DIM // 128, 128)
  t128 = jax.lax.dynamic_update_slice(t128, tail, (_NT * 32, 0))
  tlin = t128.reshape(_V, _DIM)
  out = _gather_kernel(nch)(xr, tlin)
  return out.reshape(bsz, seq, _DIM)


# speedup vs baseline: 1.4585x; 1.4578x over previous
import functools
import math

import jax
import jax.numpy as jnp
from jax import lax
from jax.experimental import pallas as pl
from jax.experimental.pallas import tpu as pltpu
from jax.experimental.pallas import tpu_sc as plsc

_DIM = 32
_EMB_SCALE = math.sqrt(float(_DIM))
_NC, _NS, _L = 2, 16, 16
_NW = _NC * _NS
_CH = 128
_NBUF = 8
_V = 1000000
_NT = _V // 128
_VTAIL = _V - _NT * 128


def _transpose_kernel():
  mesh = plsc.VectorSubcoreMesh(core_axis_name="c", subcore_axis_name="s")

  @functools.partial(
      pl.kernel,
      out_type=jax.ShapeDtypeStruct((_V // 4, 128), jnp.float32),
      mesh=mesh,
      compiler_params=pltpu.CompilerParams(use_tc_tiling_on_sc=True,
                                           needs_layout_passes=False),
      scratch_types=[
          pltpu.VMEM((2, 4, 8, 129), jnp.float32),
          pltpu.VMEM((2, 32, 128), jnp.float32),
          [pltpu.SemaphoreType.DMA] * 2,
          [pltpu.SemaphoreType.DMA] * 2,
      ],
  )
  def body(tt_hbm, out_hbm, in_v, outs_v, isems, osems):
    wid = lax.axis_index("s") * _NC + lax.axis_index("c")
    nk = (_NT - wid + _NW - 1) // _NW

    iot = lax.iota(jnp.int32, _L)
    dtv = [lax.shift_right_logical(iot, 3),
           lax.shift_right_logical(iot + 16, 3)]
    dmv = [lax.bitwise_and(iot, 7), lax.bitwise_and(iot + 16, 7)]
    clv = [jnp.zeros((_L,), jnp.int32) + c for c in range(4)]

    def _load(t, bb):
      for dt in range(4):
        pltpu.async_copy(
            tt_hbm.at[pl.ds(dt * 8, 8), pl.ds(t * 128, 128)],
            in_v.at[bb, dt, :, pl.ds(0, 128)], isems[bb])

    def _wait_load(t, bb):
      for dt in range(4):
        pltpu.make_async_copy(
            tt_hbm.at[pl.ds(dt * 8, 8), pl.ds(t * 128, 128)],
            in_v.at[bb, dt, :, pl.ds(0, 128)], isems[bb]).wait()

    def _compute(bb):
      @pl.loop(0, 32, unroll=2)
      def _lines(jj):
        jj4 = jnp.zeros((_L,), jnp.int32) + 4 * jj
        for lg in range(8):
          col = jj4 + clv[lg >> 1]
          v = plsc.load_gather(in_v.at[bb], [dtv[lg & 1], dmv[lg & 1], col])
          outs_v[bb, jj, pl.ds(lg * 16, _L)] = v * _EMB_SCALE

    @pl.when(nk > 0)
    def _prime():
      _load(wid, 0)

    @pl.loop(0, nk)
    def _tiles(k):
      t = wid + k * _NW
      for bb in range(2):
        @pl.when(lax.rem(k, 2) == bb)
        def _go():
          @pl.when(k + 1 < nk)
          def _next():
            _load(t + _NW, 1 - bb)
          _wait_load(t, bb)

          @pl.when(k >= 2)
          def _wout():
            pltpu.make_async_copy(
                outs_v.at[bb], out_hbm.at[pl.ds((t - 2 * _NW) * 32, 32)],
                osems[bb]).wait()
          _compute(bb)
          pltpu.async_copy(
              outs_v.at[bb], out_hbm.at[pl.ds(t * 32, 32)], osems[bb])

    for j in range(1, 3):
      @pl.when(nk >= j)
      def _drain():
        kl = nk - j
        tl = wid + kl * _NW
        for bb in range(2):
          @pl.when(lax.rem(kl, 2) == bb)
          def _dw():
            pltpu.make_async_copy(
                outs_v.at[bb], out_hbm.at[pl.ds(tl * 32, 32)],
                osems[bb]).wait()

  return body


def _gather_kernel(nch):
  mesh = plsc.VectorSubcoreMesh(core_axis_name="c", subcore_axis_name="s")

  @functools.partial(
      pl.kernel,
      out_type=jax.ShapeDtypeStruct((_NW * nch * _CH, _DIM), jnp.float32),
      mesh=mesh,
      compiler_params=pltpu.CompilerParams(use_tc_tiling_on_sc=False),
      scratch_types=[
          pltpu.VMEM((nch, _CH), jnp.int32),
          pltpu.VMEM((_NBUF, _CH, _DIM), jnp.float32),
          [pltpu.SemaphoreType.DMA] * _NBUF,
          [pltpu.SemaphoreType.DMA] * _NBUF,
      ],
  )
  def body(x_hbm, table_hbm, out_hbm, idx_v, rows_v, gsems, osems):
    wid = lax.axis_index("s") * _NC + lax.axis_index("c")
    och = _CH
    obase = wid * (nch * och)
    pltpu.sync_copy(x_hbm.at[wid], idx_v)

    @pl.loop(0, nch, step=_NBUF)
    def _group(g):
      for b in range(_NBUF):
        @pl.when(g > 0)
        def _drain():
          pltpu.make_async_copy(
              rows_v.at[b],
              out_hbm.at[pl.ds(obase + (g - _NBUF + b) * och, och)],
              osems[b]).wait()
        pltpu.async_copy(
            table_hbm.at[idx_v.at[g + b]], rows_v.at[b], gsems[b])
      for b in range(_NBUF):
        c = g + b
        pltpu.make_async_copy(
            table_hbm.at[idx_v.at[c]], rows_v.at[b], gsems[b]).wait()
        pltpu.async_copy(
            rows_v.at[b], out_hbm.at[pl.ds(obase + c * och, och)], osems[b])

    for b in range(_NBUF):
      pltpu.make_async_copy(
          rows_v.at[b],
          out_hbm.at[pl.ds(obase + (nch - _NBUF + b) * och, och)],
          osems[b]).wait()

  return body


def kernel(x, table):
  bsz, seq = x.shape
  tot = bsz * seq
  nch = tot // (_NW * _CH)
  xr = x.astype(jnp.int32).reshape(_NW, nch, _CH)
  t128 = _transpose_kernel()(table.T)
  tail = (table[_NT * 128:] * _EMB_SCALE).reshape(_VTAIL * _DIM // 128, 128)
  t128 = jax.lax.dynamic_update_slice(t128, tail, (_NT * 32, 0))
  tlin = t128.reshape(_V, _DIM)
  out = _gather_kernel(nch)(xr, tlin)
  return out.reshape(bsz, seq, _DIM)


# speedup vs baseline: 3.1603x; 2.1668x over previous
import functools
import math

import jax
import jax.numpy as jnp
from jax import lax
from jax.experimental import pallas as pl
from jax.experimental.pallas import tpu as pltpu
from jax.experimental.pallas import tpu_sc as plsc

_DIM = 32
_EMB_SCALE = math.sqrt(float(_DIM))
_NC, _NS, _L = 2, 16, 16
_NW = _NC * _NS
_CH = 128
_NBUF = 8
_V = 1000000
_NT = _V // 128
_VTAIL = _V - _NT * 128
_GT = 4
_BW = 128 * _GT
_SKW = _BW + 5
_NG = _NT // _GT


def _transpose_kernel():
  mesh = plsc.VectorSubcoreMesh(core_axis_name="c", subcore_axis_name="s")

  @functools.partial(
      pl.kernel,
      out_type=jax.ShapeDtypeStruct((_V // 4, 128), jnp.float32),
      mesh=mesh,
      compiler_params=pltpu.CompilerParams(use_tc_tiling_on_sc=True,
                                           needs_layout_passes=False),
      scratch_types=[
          pltpu.VMEM((2, 4, 8, _SKW), jnp.float32),
          pltpu.VMEM((2, _BW // 4, 128), jnp.float32),
          [pltpu.SemaphoreType.DMA] * 2,
          [pltpu.SemaphoreType.DMA] * 2,
      ],
  )
  def body(tt_hbm, out_hbm, in_v, outs_v, isems, osems):
    wid = lax.axis_index("s") * _NC + lax.axis_index("c")
    nk = (_NG - wid + _NW - 1) // _NW

    iot = lax.iota(jnp.int32, _L)
    dtv = [lax.shift_right_logical(iot, 3),
           lax.shift_right_logical(iot + 16, 3)]
    dmv = [lax.bitwise_and(iot, 7), lax.bitwise_and(iot + 16, 7)]
    clv = [jnp.zeros((_L,), jnp.int32) + c for c in range(4)]

    def _load(g, bb):
      for dt in range(4):
        pltpu.async_copy(
            tt_hbm.at[pl.ds(dt * 8, 8), pl.ds(g * _BW, _BW)],
            in_v.at[bb, dt, :, pl.ds(0, _BW)], isems[bb])

    def _wait_load(g, bb):
      for dt in range(4):
        pltpu.make_async_copy(
            tt_hbm.at[pl.ds(dt * 8, 8), pl.ds(g * _BW, _BW)],
            in_v.at[bb, dt, :, pl.ds(0, _BW)], isems[bb]).wait()

    def _compute(bb):
      @plsc.parallel_loop(0, _BW // 4, unroll=4)
      def _lines(jj):
        jj4 = jnp.zeros((_L,), jnp.int32) + 4 * jj
        for lg in range(8):
          col = jj4 + clv[lg >> 1]
          v = plsc.load_gather(in_v.at[bb], [dtv[lg & 1], dmv[lg & 1], col])
          outs_v[bb, jj, pl.ds(lg * 16, _L)] = v * _EMB_SCALE

    @pl.when(nk > 0)
    def _prime():
      _load(wid, 0)

    @pl.loop(0, nk)
    def _blocks(k):
      g = wid + k * _NW
      for bb in range(2):
        @pl.when(lax.rem(k, 2) == bb)
        def _go():
          @pl.when(k + 1 < nk)
          def _next():
            _load(g + _NW, 1 - bb)
          _wait_load(g, bb)

          @pl.when(k >= 2)
          def _wout():
            pltpu.make_async_copy(
                outs_v.at[bb],
                out_hbm.at[pl.ds((g - 2 * _NW) * (_BW // 4), _BW // 4)],
                osems[bb]).wait()
          _compute(bb)
          pltpu.async_copy(
              outs_v.at[bb],
              out_hbm.at[pl.ds(g * (_BW // 4), _BW // 4)], osems[bb])

    for j in range(1, 3):
      @pl.when(nk >= j)
      def _drain():
        kl = nk - j
        gl = wid + kl * _NW
        for bb in range(2):
          @pl.when(lax.rem(kl, 2) == bb)
          def _dw():
            pltpu.make_async_copy(
                outs_v.at[bb],
                out_hbm.at[pl.ds(gl * (_BW // 4), _BW // 4)],
                osems[bb]).wait()

  return body


def _gather_kernel(nch):
  mesh = plsc.VectorSubcoreMesh(core_axis_name="c", subcore_axis_name="s")

  @functools.partial(
      pl.kernel,
      out_type=jax.ShapeDtypeStruct((_NW * nch * _CH, _DIM), jnp.float32),
      mesh=mesh,
      compiler_params=pltpu.CompilerParams(use_tc_tiling_on_sc=False),
      scratch_types=[
          pltpu.VMEM((nch, _CH), jnp.int32),
          pltpu.VMEM((_NBUF, _CH, _DIM), jnp.float32),
          [pltpu.SemaphoreType.DMA] * _NBUF,
          [pltpu.SemaphoreType.DMA] * _NBUF,
      ],
  )
  def body(x_hbm, table_hbm, out_hbm, idx_v, rows_v, gsems, osems):
    wid = lax.axis_index("s") * _NC + lax.axis_index("c")
    och = _CH
    obase = wid * (nch * och)
    pltpu.sync_copy(x_hbm.at[wid], idx_v)

    @pl.loop(0, nch, step=_NBUF)
    def _group(g):
      for b in range(_NBUF):
        @pl.when(g > 0)
        def _drain():
          pltpu.make_async_copy(
              rows_v.at[b],
              out_hbm.at[pl.ds(obase + (g - _NBUF + b) * och, och)],
              osems[b]).wait()
        pltpu.async_copy(
            table_hbm.at[idx_v.at[g + b]], rows_v.at[b], gsems[b])
      for b in range(_NBUF):
        c = g + b
        pltpu.make_async_copy(
            table_hbm.at[idx_v.at[c]], rows_v.at[b], gsems[b]).wait()
        pltpu.async_copy(
            rows_v.at[b], out_hbm.at[pl.ds(obase + c * och, och)], osems[b])

    for b in range(_NBUF):
      pltpu.make_async_copy(
          rows_v.at[b],
          out_hbm.at[pl.ds(obase + (nch - _NBUF + b) * och, och)],
          osems[b]).wait()

  return body


def _format_kernel():
  def body(in_ref, out_ref):
    x = in_ref[...].reshape(128, 50, 128).reshape(128, 6400)
    out_ref[...] = x.T.reshape(200, 32, 128)

  return pl.pallas_call(
      body,
      grid=(32,),
      in_specs=[pl.BlockSpec((6400, 128), lambda i: (i, 0))],
      out_specs=pl.BlockSpec((200, 32, 128), lambda i: (0, 0, i)),
      out_shape=jax.ShapeDtypeStruct((200, 32, 4096), jnp.float32),
  )


def kernel(x, table):
  bsz, seq = x.shape
  tot = bsz * seq
  nch = tot // (_NW * _CH)
  xr = x.astype(jnp.int32).reshape(_NW, nch, _CH)
  t128 = _transpose_kernel()(table.T)
  tail = (table[_NT * 128:] * _EMB_SCALE).reshape(_VTAIL * _DIM // 128, 128)
  t128 = jax.lax.dynamic_update_slice(t128, tail, (_NT * 32, 0))
  tlin = t128.reshape(_V, _DIM)
  out = _gather_kernel(nch)(xr, tlin)
  out_t = _format_kernel()(out.reshape(bsz * seq * _DIM // 128, 128))
  return jnp.transpose(out_t, (2, 0, 1))


# speedup vs baseline: 3.3202x; 1.0506x over previous
import functools
import math

import jax
import jax.numpy as jnp
from jax import lax
from jax.experimental import pallas as pl
from jax.experimental.pallas import tpu as pltpu
from jax.experimental.pallas import tpu_sc as plsc

_DIM = 32
_EMB_SCALE = math.sqrt(float(_DIM))
_NC, _NS, _L = 2, 16, 16
_NW = _NC * _NS
_CH = 128
_NBUF = 8
_V = 1000000
_NT = _V // 128
_VTAIL = _V - _NT * 128
_GT = 1
_BW = 128 * _GT
_SKW = _BW + 1
_NG = _NT // _GT


def _transpose_kernel():
  mesh = plsc.VectorSubcoreMesh(core_axis_name="c", subcore_axis_name="s")

  @functools.partial(
      pl.kernel,
      out_type=jax.ShapeDtypeStruct((_V // 4, 128), jnp.float32),
      mesh=mesh,
      compiler_params=pltpu.CompilerParams(use_tc_tiling_on_sc=True,
                                           needs_layout_passes=False),
      scratch_types=[
          pltpu.VMEM((2, 4, 8, _SKW), jnp.float32),
          pltpu.VMEM((2, _BW // 4, 128), jnp.float32),
          [pltpu.SemaphoreType.DMA] * 2,
          [pltpu.SemaphoreType.DMA] * 2,
      ],
  )
  def body(tt_hbm, out_hbm, in_v, outs_v, isems, osems):
    wid = lax.axis_index("s") * _NC + lax.axis_index("c")
    nk = (_NG - wid + _NW - 1) // _NW

    iot = lax.iota(jnp.int32, _L)
    dtv = [lax.shift_right_logical(iot, 3),
           lax.shift_right_logical(iot + 16, 3)]
    dmv = [lax.bitwise_and(iot, 7), lax.bitwise_and(iot + 16, 7)]
    clv = [jnp.zeros((_L,), jnp.int32) + c for c in range(4)]

    def _load(g, bb):
      for dt in range(4):
        pltpu.async_copy(
            tt_hbm.at[pl.ds(dt * 8, 8), pl.ds(g * _BW, _BW)],
            in_v.at[bb, dt, :, pl.ds(0, _BW)], isems[bb])

    def _wait_load(g, bb):
      for dt in range(4):
        pltpu.make_async_copy(
            tt_hbm.at[pl.ds(dt * 8, 8), pl.ds(g * _BW, _BW)],
            in_v.at[bb, dt, :, pl.ds(0, _BW)], isems[bb]).wait()

    def _compute(bb):
      @plsc.parallel_loop(0, _BW // 4, unroll=4)
      def _lines(jj):
        jj4 = jnp.zeros((_L,), jnp.int32) + 4 * jj
        for lg in range(8):
          col = jj4 + clv[lg >> 1]
          v = plsc.load_gather(in_v.at[bb], [dtv[lg & 1], dmv[lg & 1], col])
          outs_v[bb, jj, pl.ds(lg * 16, _L)] = v * _EMB_SCALE

    @pl.when(nk > 0)
    def _prime():
      _load(wid, 0)

    @pl.loop(0, nk)
    def _blocks(k):
      g = wid + k * _NW
      for bb in range(2):
        @pl.when(lax.rem(k, 2) == bb)
        def _go():
          @pl.when(k + 1 < nk)
          def _next():
            _load(g + _NW, 1 - bb)
          _wait_load(g, bb)

          @pl.when(k >= 2)
          def _wout():
            pltpu.make_async_copy(
                outs_v.at[bb],
                out_hbm.at[pl.ds((g - 2 * _NW) * (_BW // 4), _BW // 4)],
                osems[bb]).wait()
          _compute(bb)
          pltpu.async_copy(
              outs_v.at[bb],
              out_hbm.at[pl.ds(g * (_BW // 4), _BW // 4)], osems[bb])

    for j in range(1, 3):
      @pl.when(nk >= j)
      def _drain():
        kl = nk - j
        gl = wid + kl * _NW
        for bb in range(2):
          @pl.when(lax.rem(kl, 2) == bb)
          def _dw():
            pltpu.make_async_copy(
                outs_v.at[bb],
                out_hbm.at[pl.ds(gl * (_BW // 4), _BW // 4)],
                osems[bb]).wait()

  return body


def _gather_kernel(nch):
  mesh = plsc.VectorSubcoreMesh(core_axis_name="c", subcore_axis_name="s")

  @functools.partial(
      pl.kernel,
      out_type=jax.ShapeDtypeStruct((_NW * nch * _CH, _DIM), jnp.float32),
      mesh=mesh,
      compiler_params=pltpu.CompilerParams(use_tc_tiling_on_sc=False),
      scratch_types=[
          pltpu.VMEM((nch, _CH), jnp.int32),
          pltpu.VMEM((_NBUF, _CH, _DIM), jnp.float32),
          [pltpu.SemaphoreType.DMA] * _NBUF,
          [pltpu.SemaphoreType.DMA] * _NBUF,
      ],
  )
  def body(x_hbm, table_hbm, out_hbm, idx_v, rows_v, gsems, osems):
    wid = lax.axis_index("s") * _NC + lax.axis_index("c")
    och = _CH
    obase = wid * (nch * och)
    pltpu.sync_copy(x_hbm.at[wid], idx_v)

    @pl.loop(0, nch, step=_NBUF)
    def _group(g):
      for b in range(_NBUF):
        @pl.when(g > 0)
        def _drain():
          pltpu.make_async_copy(
              rows_v.at[b],
              out_hbm.at[pl.ds(obase + (g - _NBUF + b) * och, och)],
              osems[b]).wait()
        pltpu.async_copy(
            table_hbm.at[idx_v.at[g + b]], rows_v.at[b], gsems[b])
      for b in range(_NBUF):
        c = g + b
        pltpu.make_async_copy(
            table_hbm.at[idx_v.at[c]], rows_v.at[b], gsems[b]).wait()
        pltpu.async_copy(
            rows_v.at[b], out_hbm.at[pl.ds(obase + c * och, och)], osems[b])

    for b in range(_NBUF):
      pltpu.make_async_copy(
          rows_v.at[b],
          out_hbm.at[pl.ds(obase + (nch - _NBUF + b) * och, och)],
          osems[b]).wait()

  return body


def _format_kernel():
  def body(in_ref, out_ref):
    x = in_ref[...].reshape(128, 50, 128).reshape(128, 6400)
    out_ref[...] = x.T.reshape(200, 32, 128)

  return pl.pallas_call(
      body,
      grid=(32,),
      in_specs=[pl.BlockSpec((6400, 128), lambda i: (i, 0))],
      out_specs=pl.BlockSpec((200, 32, 128), lambda i: (0, 0, i)),
      out_shape=jax.ShapeDtypeStruct((200, 32, 4096), jnp.float32),
  )


def kernel(x, table):
  bsz, seq = x.shape
  tot = bsz * seq
  nch = tot // (_NW * _CH)
  xr = x.astype(jnp.int32).reshape(_NW, nch, _CH)
  t128 = _transpose_kernel()(table.T)
  tail = (table[_NT * 128:] * _EMB_SCALE).reshape(_VTAIL * _DIM // 128, 128)
  t128 = jax.lax.dynamic_update_slice(t128, tail, (_NT * 32, 0))
  tlin = t128.reshape(_V, _DIM)
  out = _gather_kernel(nch)(xr, tlin)
  out_t = _format_kernel()(out.reshape(bsz * seq * _DIM // 128, 128))
  return jnp.transpose(out_t, (2, 0, 1))
